# baseline (speedup 1.0000x reference)
import functools

import jax
import jax.numpy as jnp
from jax import lax
from jax.experimental import pallas as pl
from jax.experimental.pallas import tpu as pltpu
from jax.experimental.pallas import tpu_sc as plsc

_B = 640
_FB = 2048
_BIG = 10 ** 6


def _gate_body(x_ref, gw_ref, gb_ref, u_ref, u8_ref,
               ent_ref, wc_ref, pos_ref, ast_ref, *, n_tok, n_exp):
    logits = jnp.dot(x_ref[...], gw_ref[...],
                     preferred_element_type=jnp.float32) + gb_ref[...]
    lane = lax.broadcasted_iota(jnp.int32, logits.shape, 1)
    neg = jnp.float32(-1e30)
    ml = jnp.where(lane < n_exp, logits, neg)

    m = jnp.max(ml, axis=1, keepdims=True)
    ex = jnp.exp(ml - m)
    p = ex / jnp.sum(ex, axis=1, keepdims=True)
    ent = -jnp.sum(p * jnp.log(p + 1e-8), axis=1)
    ent_ref[...] = jnp.full(ent_ref.shape, jnp.sum(ent) / n_tok, jnp.float32)

    i1 = jnp.min(jnp.where(ml == m, lane, _BIG), axis=1, keepdims=True)
    ml2 = jnp.where(lane == i1, neg, ml)
    m2 = jnp.max(ml2, axis=1, keepdims=True)
    i2 = jnp.min(jnp.where(ml2 == m2, lane, _BIG), axis=1, keepdims=True)

    w1 = 1.0 / (1.0 + jnp.exp(m2 - m))
    w2 = 1.0 - w1
    wc_ref[...] = jnp.where(lane == 0, w1, 0.0) + jnp.where(lane == 1, w2, 0.0)

    ohf = ((lane == i1) | (lane == i2)).astype(jnp.float32)
    exc = jnp.dot(u_ref[...], ohf, preferred_element_type=jnp.float32)
    cnt = exc[n_tok - 1:n_tok, :] + ohf[n_tok - 1:n_tok, :]
    r = ((cnt.astype(jnp.int32) + (_B - 1)) // _B) * _B
    astart = jnp.dot(r.astype(jnp.float32), u8_ref[...],
                     preferred_element_type=jnp.float32)
    ast_ref[...] = jnp.broadcast_to(astart.astype(jnp.int32), ast_ref.shape)

    pos_all = astart + exc
    pos0 = jnp.sum(jnp.where(lane == i1, pos_all, 0.0), axis=1, keepdims=True)
    pos1 = jnp.sum(jnp.where(lane == i2, pos_all, 0.0), axis=1, keepdims=True)
    pos_ref[...] = (jnp.where(lane == 0, pos0, 0.0) +
                    jnp.where(lane == 1, pos1, 0.0)).astype(jnp.int32)


def _gmm_body(m_ref, xs_ref, w1_ref, b1_ref, w2_ref, b2_ref, ys_ref, acc_ref,
              *, nj, smax):
    s = pl.program_id(0)
    j = pl.program_id(1)
    real = s < m_ref[smax]

    @pl.when(real & (j == 0))
    def _():
        acc_ref[...] = jnp.zeros_like(acc_ref)

    @pl.when(real)
    def _():
        h = jnp.dot(xs_ref[...], w1_ref[0],
                    preferred_element_type=jnp.float32) + b1_ref[0]
        h = jnp.maximum(h, 0.0)
        acc_ref[...] += jnp.dot(h, w2_ref[0],
                                preferred_element_type=jnp.float32)

    @pl.when(real & (j == nj - 1))
    def _():
        ys_ref[...] = acc_ref[...] + b2_ref[0]


def _comb_body(y0_ref, y1_ref, wc_ref, out_ref):
    out_ref[...] = (y0_ref[...] * wc_ref[:, 0:1] +
                    y1_ref[...] * wc_ref[:, 1:2])


def _sc_dispatch(x, pos0, pos1, p_rows):
    n_tok, d = x.shape
    info = plsc.get_sparse_core_info()
    nw = info.num_cores * info.num_subcores
    tpn = n_tok // nw
    mesh = plsc.VectorSubcoreMesh(core_axis_name="c", subcore_axis_name="s")

    @functools.partial(
        pl.kernel,
        out_type=jax.ShapeDtypeStruct((p_rows, d), jnp.float32),
        mesh=mesh,
        scratch_types=[
            pltpu.VMEM((tpn, d), jnp.float32),
            pltpu.VMEM((tpn,), jnp.int32),
            pltpu.VMEM((tpn,), jnp.int32),
            pltpu.SemaphoreType.DMA,
            pltpu.SemaphoreType.DMA,
        ],
    )
    def k(x_hbm, p0_hbm, p1_hbm, xs_hbm, xv, p0v, p1v, sem0, sem1):
        wid = lax.axis_index("s") * info.num_cores + lax.axis_index("c")
        base = wid * tpn
        pltpu.sync_copy(x_hbm.at[pl.ds(base, tpn)], xv)
        pltpu.sync_copy(p0_hbm.at[pl.ds(base, tpn)], p0v)
        pltpu.sync_copy(p1_hbm.at[pl.ds(base, tpn)], p1v)
        c0 = pltpu.async_copy(xv, xs_hbm.at[p0v], sem0)
        c1 = pltpu.async_copy(xv, xs_hbm.at[p1v], sem1)
        c0.wait()
        c1.wait()

    return k(x, pos0, pos1)


def _sc_gather(ys, pos0, pos1, n_tok):
    _, d = ys.shape
    info = plsc.get_sparse_core_info()
    nw = info.num_cores * info.num_subcores
    tpn = n_tok // nw
    ch = tpn // 2
    mesh = plsc.VectorSubcoreMesh(core_axis_name="c", subcore_axis_name="s")

    @functools.partial(
        pl.kernel,
        out_type=[jax.ShapeDtypeStruct((n_tok, d), jnp.float32),
                  jax.ShapeDtypeStruct((n_tok, d), jnp.float32)],
        mesh=mesh,
        scratch_types=[
            pltpu.VMEM((ch, d), jnp.float32),
            pltpu.VMEM((ch, d), jnp.float32),
            pltpu.VMEM((ch,), jnp.int32),
            pltpu.VMEM((ch,), jnp.int32),
            pltpu.SemaphoreType.DMA,
            pltpu.SemaphoreType.DMA,
        ],
    )
    def k(ys_hbm, p0_hbm, p1_hbm, y0_hbm, y1_hbm, y0v, y1v, p0v, p1v, s0, s1):
        wid = lax.axis_index("s") * info.num_cores + lax.axis_index("c")
        for c in range(tpn // ch):
            base = wid * tpn + c * ch
            pltpu.sync_copy(p0_hbm.at[pl.ds(base, ch)], p0v)
            pltpu.sync_copy(p1_hbm.at[pl.ds(base, ch)], p1v)
            c0 = pltpu.async_copy(ys_hbm.at[p0v], y0v, s0)
            c1 = pltpu.async_copy(ys_hbm.at[p1v], y1v, s1)
            c0.wait()
            c1.wait()
            pltpu.sync_copy(y0v, y0_hbm.at[pl.ds(base, ch)])
            pltpu.sync_copy(y1v, y1_hbm.at[pl.ds(base, ch)])

    return k(ys, pos0, pos1)


def kernel(x, gate_w, gate_b, W1, b1, W2, b2):
    n_tok, d = x.shape
    n_exp = gate_w.shape[1]
    f = W1.shape[2]
    nj = f // _FB
    smax = -((-2 * n_tok) // _B) + n_exp - 1
    p_rows = smax * _B

    gwp = jnp.pad(gate_w, ((0, 0), (0, 128 - n_exp)))
    gbp = jnp.pad(gate_b, (0, 128 - n_exp)).reshape(1, 128)
    u = jnp.tri(n_tok, k=-1, dtype=jnp.float32)
    u8 = jnp.triu(jnp.ones((128, 128), jnp.float32), k=1)

    ent, wc, posout, astout = pl.pallas_call(
        functools.partial(_gate_body, n_tok=n_tok, n_exp=n_exp),
        out_shape=[
            jax.ShapeDtypeStruct((8, 128), jnp.float32),
            jax.ShapeDtypeStruct((n_tok, 128), jnp.float32),
            jax.ShapeDtypeStruct((n_tok, 128), jnp.int32),
            jax.ShapeDtypeStruct((8, 128), jnp.int32),
        ],
    )(x, gwp, gbp, u, u8)

    pos0 = posout[:, 0]
    pos1 = posout[:, 1]
    nused = astout[0, n_exp] // _B
    sblk = jnp.arange(smax, dtype=jnp.int32)
    eid = jnp.sum((astout[0, :n_exp][None, :] <= sblk[:, None] * _B)
                  .astype(jnp.int32), axis=1) - 1
    meta = jnp.concatenate([eid, nused[None]])

    xs = _sc_dispatch(x, pos0, pos1, p_rows)

    b1r = b1.reshape(n_exp, 1, f)
    b2r = b2.reshape(n_exp, 1, d)
    grid_spec = pltpu.PrefetchScalarGridSpec(
        num_scalar_prefetch=1,
        grid=(smax, nj),
        in_specs=[
            pl.BlockSpec((_B, d), lambda s, j, m: (jnp.minimum(s, m[smax] - 1), 0)),
            pl.BlockSpec((1, d, _FB),
                         lambda s, j, m: (m[s], 0, jnp.where(s >= m[smax], nj - 1, j))),
            pl.BlockSpec((1, 1, _FB),
                         lambda s, j, m: (m[s], 0, jnp.where(s >= m[smax], nj - 1, j))),
            pl.BlockSpec((1, _FB, d),
                         lambda s, j, m: (m[s], jnp.where(s >= m[smax], nj - 1, j), 0)),
            pl.BlockSpec((1, 1, d), lambda s, j, m: (m[s], 0, 0)),
        ],
        out_specs=pl.BlockSpec((_B, d),
                               lambda s, j, m: (jnp.minimum(s, m[smax] - 1), 0)),
        scratch_shapes=[pltpu.VMEM((_B, d), jnp.float32)],
    )
    ys = pl.pallas_call(
        functools.partial(_gmm_body, nj=nj, smax=smax),
        grid_spec=grid_spec,
        out_shape=jax.ShapeDtypeStruct((p_rows, d), jnp.float32),
    )(meta, xs, W1, b1r, W2, b2r)

    y0, y1 = _sc_gather(ys, pos0, pos1, n_tok)

    out = pl.pallas_call(
        _comb_body,
        out_shape=jax.ShapeDtypeStruct((n_tok, d), jnp.float32),
    )(y0, y1, wc)

    return (out, ent[0, 0])


# speedup vs baseline: 1.0433x; 1.0433x over previous
import functools

import jax
import jax.numpy as jnp
from jax import lax
from jax.experimental import pallas as pl
from jax.experimental.pallas import tpu as pltpu
from jax.experimental.pallas import tpu_sc as plsc

_B = 576
_FB = 2048
_BIG = 10 ** 6


def _gate_body(x_ref, gw_ref, gb_ref, u_ref, u8_ref,
               ent_ref, wc_ref, pos_ref, ast_ref, *, n_tok, n_exp):
    logits = jnp.dot(x_ref[...], gw_ref[...],
                     preferred_element_type=jnp.float32) + gb_ref[...]
    lane = lax.broadcasted_iota(jnp.int32, logits.shape, 1)
    neg = jnp.float32(-1e30)
    ml = jnp.where(lane < n_exp, logits, neg)

    m = jnp.max(ml, axis=1, keepdims=True)
    ex = jnp.exp(ml - m)
    p = ex / jnp.sum(ex, axis=1, keepdims=True)
    ent = -jnp.sum(p * jnp.log(p + 1e-8), axis=1)
    ent_ref[...] = jnp.full(ent_ref.shape, jnp.sum(ent) / n_tok, jnp.float32)

    i1 = jnp.min(jnp.where(ml == m, lane, _BIG), axis=1, keepdims=True)
    ml2 = jnp.where(lane == i1, neg, ml)
    m2 = jnp.max(ml2, axis=1, keepdims=True)
    i2 = jnp.min(jnp.where(ml2 == m2, lane, _BIG), axis=1, keepdims=True)

    w1 = 1.0 / (1.0 + jnp.exp(m2 - m))
    w2 = 1.0 - w1
    wc_ref[...] = jnp.where(lane == 0, w1, 0.0) + jnp.where(lane == 1, w2, 0.0)

    ohf = ((lane == i1) | (lane == i2)).astype(jnp.bfloat16)
    exc = jnp.dot(u_ref[...], ohf, preferred_element_type=jnp.float32)
    ohf = ohf.astype(jnp.float32)
    cnt = exc[n_tok - 1:n_tok, :] + ohf[n_tok - 1:n_tok, :]
    r = ((cnt.astype(jnp.int32) + (_B - 1)) // _B) * _B
    astart = jnp.dot(r.astype(jnp.float32), u8_ref[...],
                     preferred_element_type=jnp.float32)
    ast_ref[...] = jnp.broadcast_to(astart.astype(jnp.int32), ast_ref.shape)

    pos_all = astart + exc
    pos0 = jnp.sum(jnp.where(lane == i1, pos_all, 0.0), axis=1, keepdims=True)
    pos1 = jnp.sum(jnp.where(lane == i2, pos_all, 0.0), axis=1, keepdims=True)
    pos_ref[...] = (jnp.where(lane == 0, pos0, 0.0) +
                    jnp.where(lane == 1, pos1, 0.0)).astype(jnp.int32)


def _gmm_body(m_ref, xs_ref, w1_ref, b1_ref, w2_ref, b2_ref, ys_ref, acc_ref,
              *, nj, smax):
    s = pl.program_id(0)
    j = pl.program_id(1)
    real = s < m_ref[smax]

    @pl.when(real & (j == 0))
    def _():
        acc_ref[...] = jnp.zeros_like(acc_ref)

    @pl.when(real)
    def _():
        h = jnp.dot(xs_ref[...], w1_ref[0],
                    preferred_element_type=jnp.float32) + b1_ref[0]
        h = jnp.maximum(h, 0.0)
        acc_ref[...] += jnp.dot(h, w2_ref[0],
                                preferred_element_type=jnp.float32)

    @pl.when(real & (j == nj - 1))
    def _():
        ys_ref[...] = acc_ref[...] + b2_ref[0]


def _comb_body(y0_ref, y1_ref, wc_ref, out_ref):
    out_ref[...] = (y0_ref[...] * wc_ref[:, 0:1] +
                    y1_ref[...] * wc_ref[:, 1:2])


def _sc_dispatch(x, pos0, pos1, p_rows):
    n_tok, d = x.shape
    info = plsc.get_sparse_core_info()
    nw = info.num_cores * info.num_subcores
    tpn = n_tok // nw
    mesh = plsc.VectorSubcoreMesh(core_axis_name="c", subcore_axis_name="s")

    @functools.partial(
        pl.kernel,
        out_type=jax.ShapeDtypeStruct((p_rows, d), jnp.float32),
        mesh=mesh,
        scratch_types=[
            pltpu.VMEM((tpn, d), jnp.float32),
            pltpu.VMEM((tpn,), jnp.int32),
            pltpu.VMEM((tpn,), jnp.int32),
            pltpu.SemaphoreType.DMA,
            pltpu.SemaphoreType.DMA,
        ],
    )
    def k(x_hbm, p0_hbm, p1_hbm, xs_hbm, xv, p0v, p1v, sem0, sem1):
        wid = lax.axis_index("s") * info.num_cores + lax.axis_index("c")
        base = wid * tpn
        pltpu.sync_copy(x_hbm.at[pl.ds(base, tpn)], xv)
        pltpu.sync_copy(p0_hbm.at[pl.ds(base, tpn)], p0v)
        pltpu.sync_copy(p1_hbm.at[pl.ds(base, tpn)], p1v)
        c0 = pltpu.async_copy(xv, xs_hbm.at[p0v], sem0)
        c1 = pltpu.async_copy(xv, xs_hbm.at[p1v], sem1)
        c0.wait()
        c1.wait()

    return k(x, pos0, pos1)


def _sc_gather(ys, pos0, pos1, n_tok):
    _, d = ys.shape
    info = plsc.get_sparse_core_info()
    nw = info.num_cores * info.num_subcores
    tpn = n_tok // nw
    ch = tpn // 2
    mesh = plsc.VectorSubcoreMesh(core_axis_name="c", subcore_axis_name="s")

    @functools.partial(
        pl.kernel,
        out_type=[jax.ShapeDtypeStruct((n_tok, d), jnp.float32),
                  jax.ShapeDtypeStruct((n_tok, d), jnp.float32)],
        mesh=mesh,
        scratch_types=[
            pltpu.VMEM((ch, d), jnp.float32),
            pltpu.VMEM((ch, d), jnp.float32),
            pltpu.VMEM((ch,), jnp.int32),
            pltpu.VMEM((ch,), jnp.int32),
            pltpu.SemaphoreType.DMA,
            pltpu.SemaphoreType.DMA,
        ],
    )
    def k(ys_hbm, p0_hbm, p1_hbm, y0_hbm, y1_hbm, y0v, y1v, p0v, p1v, s0, s1):
        wid = lax.axis_index("s") * info.num_cores + lax.axis_index("c")
        for c in range(tpn // ch):
            base = wid * tpn + c * ch
            pltpu.sync_copy(p0_hbm.at[pl.ds(base, ch)], p0v)
            pltpu.sync_copy(p1_hbm.at[pl.ds(base, ch)], p1v)
            c0 = pltpu.async_copy(ys_hbm.at[p0v], y0v, s0)
            c1 = pltpu.async_copy(ys_hbm.at[p1v], y1v, s1)
            c0.wait()
            c1.wait()
            pltpu.sync_copy(y0v, y0_hbm.at[pl.ds(base, ch)])
            pltpu.sync_copy(y1v, y1_hbm.at[pl.ds(base, ch)])

    return k(ys, pos0, pos1)


def kernel(x, gate_w, gate_b, W1, b1, W2, b2):
    n_tok, d = x.shape
    n_exp = gate_w.shape[1]
    f = W1.shape[2]
    nj = f // _FB
    smax = -((-2 * n_tok) // _B) + n_exp - 1
    p_rows = smax * _B

    gwp = jnp.pad(gate_w, ((0, 0), (0, 128 - n_exp)))
    gbp = jnp.pad(gate_b, (0, 128 - n_exp)).reshape(1, 128)
    u = jnp.tri(n_tok, k=-1, dtype=jnp.bfloat16)
    u8 = jnp.triu(jnp.ones((128, 128), jnp.float32), k=1)

    ent, wc, posout, astout = pl.pallas_call(
        functools.partial(_gate_body, n_tok=n_tok, n_exp=n_exp),
        out_shape=[
            jax.ShapeDtypeStruct((8, 128), jnp.float32),
            jax.ShapeDtypeStruct((n_tok, 128), jnp.float32),
            jax.ShapeDtypeStruct((n_tok, 128), jnp.int32),
            jax.ShapeDtypeStruct((8, 128), jnp.int32),
        ],
    )(x, gwp, gbp, u, u8)

    pos0 = posout[:, 0]
    pos1 = posout[:, 1]
    nused = astout[0, n_exp] // _B
    sblk = jnp.arange(smax, dtype=jnp.int32)
    eid = jnp.sum((astout[0, :n_exp][None, :] <= sblk[:, None] * _B)
                  .astype(jnp.int32), axis=1) - 1
    meta = jnp.concatenate([eid, nused[None]])

    xs = _sc_dispatch(x, pos0, pos1, p_rows)

    b1r = b1.reshape(n_exp, 1, f)
    b2r = b2.reshape(n_exp, 1, d)
    grid_spec = pltpu.PrefetchScalarGridSpec(
        num_scalar_prefetch=1,
        grid=(smax, nj),
        in_specs=[
            pl.BlockSpec((_B, d), lambda s, j, m: (jnp.minimum(s, m[smax] - 1), 0)),
            pl.BlockSpec((1, d, _FB),
                         lambda s, j, m: (m[s], 0, jnp.where(s >= m[smax], nj - 1, j))),
            pl.BlockSpec((1, 1, _FB),
                         lambda s, j, m: (m[s], 0, jnp.where(s >= m[smax], nj - 1, j))),
            pl.BlockSpec((1, _FB, d),
                         lambda s, j, m: (m[s], jnp.where(s >= m[smax], nj - 1, j), 0)),
            pl.BlockSpec((1, 1, d), lambda s, j, m: (m[s], 0, 0)),
        ],
        out_specs=pl.BlockSpec((_B, d),
                               lambda s, j, m: (jnp.minimum(s, m[smax] - 1), 0)),
        scratch_shapes=[pltpu.VMEM((_B, d), jnp.float32)],
    )
    ys = pl.pallas_call(
        functools.partial(_gmm_body, nj=nj, smax=smax),
        grid_spec=grid_spec,
        out_shape=jax.ShapeDtypeStruct((p_rows, d), jnp.float32),
    )(meta, xs, W1, b1r, W2, b2r)

    y0, y1 = _sc_gather(ys, pos0, pos1, n_tok)

    nb = n_tok // 256
    out = pl.pallas_call(
        _comb_body,
        grid=(nb,),
        in_specs=[
            pl.BlockSpec((256, d), lambda i: (i, 0)),
            pl.BlockSpec((256, d), lambda i: (i, 0)),
            pl.BlockSpec((256, 128), lambda i: (i, 0)),
        ],
        out_specs=pl.BlockSpec((256, d), lambda i: (i, 0)),
        out_shape=jax.ShapeDtypeStruct((n_tok, d), jnp.float32),
    )(y0, y1, wc)

    return (out, ent[0, 0])
